# baseline (speedup 1.0000x reference)
import functools

import jax
import jax.numpy as jnp
from jax.experimental import pallas as pl
from jax.experimental.pallas import tpu as pltpu

NC_PAD = 128
KP = 16
G = 1


def _conv_gap_fc_kernel(yt_ref, w_ref, cb_ref, fcw_ref, fcb_ref,
                        feat_ref, logits_ref, *, g, h, w_out, inv_hw):
    hw = h * w_out
    for i in range(g):
        yt = yt_ref[i]
        lhs = jnp.concatenate(
            [yt[:, 0:hw], yt[:, w_out:w_out + hw], yt[:, 2 * w_out:2 * w_out + hw]],
            axis=0)
        acc = jax.lax.dot_general(
            lhs, w_ref[...], (((0,), (0,)), ((), ())),
            preferred_element_type=jnp.float32)
        relu = jnp.maximum(acc + cb_ref[...], 0.0)
        feat_ref[i] = relu.astype(jnp.bfloat16).reshape(h, w_out, -1)
        pooled = (jnp.sum(relu, axis=0, keepdims=True) * inv_hw)
        logits = jnp.dot(pooled.astype(jnp.bfloat16), fcw_ref[...],
                         preferred_element_type=jnp.float32)
        logits_ref[i] = logits + fcb_ref[...]


@jax.jit
def _forward(x_nchw, conv_w, conv_b, fc_w, fc_b):
    B, Cin, H, W = x_nchw.shape
    Cout = conv_w.shape[0]
    NC = fc_w.shape[1]
    K = 3 * Cin

    x_p = jnp.pad(x_nchw.astype(jnp.bfloat16), ((0, 0), (0, 0), (1, 1), (1, 1)))
    slabs = [x_p[:, c, :, kx:kx + W] for kx in range(3) for c in range(Cin)]
    yt = jnp.stack(slabs, axis=1).reshape(B, K, (H + 2) * W)
    yt = jnp.pad(yt, ((0, 0), (0, KP - K), (0, 0)))

    w_rows = jnp.transpose(conv_w, (2, 3, 1, 0)).reshape(3, K, Cout)
    w_rows = jnp.pad(w_rows, ((0, 0), (0, KP - K), (0, 0)))
    w_rows = w_rows.reshape(3 * KP, Cout).astype(jnp.bfloat16)
    cb = conv_b.reshape(1, Cout).astype(jnp.float32)
    fcw = jnp.pad(fc_w, ((0, 0), (0, NC_PAD - NC))).astype(jnp.bfloat16)
    fcb = jnp.pad(fc_b, (0, NC_PAD - NC)).reshape(1, NC_PAD).astype(jnp.float32)

    body = functools.partial(_conv_gap_fc_kernel, g=G, h=H, w_out=W,
                             inv_hw=1.0 / float(H * W))
    feat, logits_pad = pl.pallas_call(
        body,
        out_shape=(
            jax.ShapeDtypeStruct((B, H, W, Cout), jnp.bfloat16),
            jax.ShapeDtypeStruct((B, 1, NC_PAD), jnp.float32),
        ),
        grid=(B // G,),
        in_specs=[
            pl.BlockSpec((G, KP, (H + 2) * W), lambda b: (b, 0, 0)),
            pl.BlockSpec((3 * KP, Cout), lambda b: (0, 0)),
            pl.BlockSpec((1, Cout), lambda b: (0, 0)),
            pl.BlockSpec((Cout, NC_PAD), lambda b: (0, 0)),
            pl.BlockSpec((1, NC_PAD), lambda b: (0, 0)),
        ],
        out_specs=(
            pl.BlockSpec((G, H, W, Cout), lambda b: (b, 0, 0, 0)),
            pl.BlockSpec((G, 1, NC_PAD), lambda b: (b, 0, 0)),
        ),
        compiler_params=pltpu.CompilerParams(
            dimension_semantics=("parallel",),
        ),
    )(yt, w_rows, cb, fcw, fcb)

    logits = logits_pad.reshape(B, NC_PAD)[:, :NC]
    return feat, logits


def kernel(x_nchw, conv_w, conv_b, fc_w, fc_b):
    return _forward(x_nchw, conv_w, conv_b, fc_w, fc_b)


# speedup vs baseline: 1.1618x; 1.1618x over previous
import functools

import jax
import jax.numpy as jnp
from jax.experimental import pallas as pl
from jax.experimental.pallas import tpu as pltpu

NC_PAD = 128
KP = 16
G = 8


def _conv_gap_fc_kernel(yt_ref, w_ref, cb_ref, fcw_ref, fcb_ref,
                        feat_ref, logits_ref, *, g, h, w_out, inv_hw):
    hw = h * w_out
    for i in range(g):
        yt = yt_ref[i]
        lhs = jnp.concatenate(
            [yt[:, 0:hw], yt[:, w_out:w_out + hw], yt[:, 2 * w_out:2 * w_out + hw]],
            axis=0)
        acc = jax.lax.dot_general(
            lhs, w_ref[...], (((0,), (0,)), ((), ())),
            preferred_element_type=jnp.float32)
        relu = jnp.maximum(acc + cb_ref[...], 0.0)
        feat_ref[i] = relu.astype(jnp.bfloat16).reshape(h, w_out, -1)
        pooled = (jnp.sum(relu, axis=0, keepdims=True) * inv_hw)
        logits = jnp.dot(pooled.astype(jnp.bfloat16), fcw_ref[...],
                         preferred_element_type=jnp.float32)
        logits_ref[i] = logits + fcb_ref[...]


@jax.jit
def _forward(x_nchw, conv_w, conv_b, fc_w, fc_b):
    B, Cin, H, W = x_nchw.shape
    Cout = conv_w.shape[0]
    NC = fc_w.shape[1]
    K = 3 * Cin

    x_p = jnp.pad(x_nchw.astype(jnp.bfloat16), ((0, 0), (0, 0), (1, 1), (1, 1)))
    slabs = [x_p[:, c, :, kx:kx + W] for kx in range(3) for c in range(Cin)]
    yt = jnp.stack(slabs, axis=1).reshape(B, K, (H + 2) * W)
    yt = jnp.pad(yt, ((0, 0), (0, KP - K), (0, 0)))

    w_rows = jnp.transpose(conv_w, (2, 3, 1, 0)).reshape(3, K, Cout)
    w_rows = jnp.pad(w_rows, ((0, 0), (0, KP - K), (0, 0)))
    w_rows = w_rows.reshape(3 * KP, Cout).astype(jnp.bfloat16)
    cb = conv_b.reshape(1, Cout).astype(jnp.float32)
    fcw = jnp.pad(fc_w, ((0, 0), (0, NC_PAD - NC))).astype(jnp.bfloat16)
    fcb = jnp.pad(fc_b, (0, NC_PAD - NC)).reshape(1, NC_PAD).astype(jnp.float32)

    body = functools.partial(_conv_gap_fc_kernel, g=G, h=H, w_out=W,
                             inv_hw=1.0 / float(H * W))
    feat, logits_pad = pl.pallas_call(
        body,
        out_shape=(
            jax.ShapeDtypeStruct((B, H, W, Cout), jnp.bfloat16),
            jax.ShapeDtypeStruct((B, 1, NC_PAD), jnp.float32),
        ),
        grid=(B // G,),
        in_specs=[
            pl.BlockSpec((G, KP, (H + 2) * W), lambda b: (b, 0, 0)),
            pl.BlockSpec((3 * KP, Cout), lambda b: (0, 0)),
            pl.BlockSpec((1, Cout), lambda b: (0, 0)),
            pl.BlockSpec((Cout, NC_PAD), lambda b: (0, 0)),
            pl.BlockSpec((1, NC_PAD), lambda b: (0, 0)),
        ],
        out_specs=(
            pl.BlockSpec((G, H, W, Cout), lambda b: (b, 0, 0, 0)),
            pl.BlockSpec((G, 1, NC_PAD), lambda b: (b, 0, 0)),
        ),
        compiler_params=pltpu.CompilerParams(
            dimension_semantics=("parallel",),
        ),
    )(yt, w_rows, cb, fcw, fcb)

    logits = logits_pad.reshape(B, NC_PAD)[:, :NC]
    return feat, logits


def kernel(x_nchw, conv_w, conv_b, fc_w, fc_b):
    return _forward(x_nchw, conv_w, conv_b, fc_w, fc_b)
